# baseline (speedup 1.0000x reference)
import functools

import jax
import jax.numpy as jnp
from jax import lax
from jax.experimental import pallas as pl
from jax.experimental.pallas import tpu as pltpu
from jax.experimental.pallas import tpu_sc as plsc

_LANES = 16
_WINDOW = 512


def _transpose_table(tbl4, rows, d):
    tj_total = tbl4.shape[1]
    mesh = plsc.VectorSubcoreMesh(
        core_axis_name="core", subcore_axis_name="subcore"
    )
    n_workers = 32
    full_rounds = tj_total // n_workers
    tail = tj_total - full_rounds * n_workers

    @functools.partial(
        pl.kernel,
        out_type=jax.ShapeDtypeStruct((rows, d), jnp.float32),
        mesh=mesh,
        scratch_types=[
            pltpu.VMEM((2, 8, 8, 128), jnp.float32),
            pltpu.VMEM((2, 128, 64), jnp.float32),
            pltpu.SemaphoreType.DMA,
            pltpu.SemaphoreType.DMA,
            pltpu.SemaphoreType.DMA,
            pltpu.SemaphoreType.DMA,
        ],
        compiler_params=pltpu.CompilerParams(needs_layout_passes=False),
    )
    def transpose_kernel(tbl4_hbm, out_hbm, vin, vout,
                         sem_i0, sem_i1, sem_o0, sem_o1):
        w = lax.axis_index("subcore") * 2 + lax.axis_index("core")
        rows16 = [k * _LANES + lax.iota(jnp.int32, _LANES) for k in range(8)]

        n_steps = full_rounds + (1 if tail else 0)
        assert n_steps % 2 == 0
        sem_i = [sem_i0, sem_i1]
        sem_o = [sem_o0, sem_o1]

        def tj_of(t):
            return t * n_workers + w

        def active(t):
            return (t < full_rounds) | (w < tail)

        def start_in(t, slot):
            pltpu.async_copy(tbl4_hbm.at[:, tj_of(t)], vin.at[slot],
                             sem_i[slot])

        def wait_in(t, slot):
            pltpu.make_async_copy(
                tbl4_hbm.at[:, tj_of(t)], vin.at[slot], sem_i[slot]
            ).wait()

        def start_out(t, slot):
            pltpu.async_copy(
                vout.at[slot],
                out_hbm.at[pl.ds(tj_of(t) * 128, 128), :],
                sem_o[slot],
            )

        def wait_out(t, slot):
            pltpu.make_async_copy(
                vout.at[slot],
                out_hbm.at[pl.ds(tj_of(t) * 128, 128), :],
                sem_o[slot],
            ).wait()

        def compute(slot):
            @pl.loop(0, 64)
            def _(dd):
                a = dd // 8
                c = lax.rem(dd, 8)
                dsplat = jnp.broadcast_to(dd, (_LANES,))
                for k in range(8):
                    vals = vin[slot, a, c, pl.ds(k * _LANES, _LANES)]
                    plsc.store_scatter(
                        vout.at[slot], [rows16[k], dsplat], vals
                    )

        start_in(0, 0)

        @pl.loop(0, n_steps // 2)
        def _(u):
            t0 = 2 * u
            t1 = 2 * u + 1

            wait_in(t0, 0)

            @pl.when(active(t1))
            def _():
                start_in(t1, 1)

            @pl.when(u >= 1)
            def _():
                wait_out(t0 - 2, 0)

            compute(0)
            start_out(t0, 0)

            @pl.when(active(t1))
            def _():
                wait_in(t1, 1)

                @pl.when((t1 + 1 < n_steps) & active(t1 + 1))
                def _():
                    start_in(t1 + 1, 0)

                @pl.when(u >= 1)
                def _():
                    wait_out(t1 - 2, 1)

                compute(1)
                start_out(t1, 1)

        wait_out(n_steps - 2, 0)

        @pl.when(active(n_steps - 1))
        def _():
            wait_out(n_steps - 1, 1)

    return transpose_kernel(tbl4)


def kernel(input_ids, table):
    B, S, H = input_ids.shape
    D = table.shape[-1]
    n_per_head = table.shape[0] // H
    N = B * S * H
    ids_flat = input_ids.reshape(1, N)

    tbl4 = jnp.transpose(
        table.reshape(table.shape[0] // 128, 128, 8, 8), (2, 0, 3, 1)
    )
    ttable = _transpose_table(tbl4, table.shape[0], D)

    mesh = plsc.VectorSubcoreMesh(
        core_axis_name="core", subcore_axis_name="subcore"
    )

    @functools.partial(
        pl.kernel,
        out_type=jax.ShapeDtypeStruct((N, D), jnp.float32),
        mesh=mesh,
        scratch_types=[pltpu.VMEM((_WINDOW,), jnp.int32)],
        compiler_params=pltpu.CompilerParams(use_tc_tiling_on_sc=False),
    )
    def gather_kernel(ids_hbm, table_hbm, out_hbm, sidx):
        def body(i_vmem, o_vmem):
            offs = (
                lax.rem(
                    lax.iota(jnp.int32, _LANES),
                    jnp.full((_LANES,), H, jnp.int32),
                )
                * n_per_head
            )
            src = i_vmem.at[0]

            @pl.loop(0, _WINDOW, step=_LANES)
            def _(j):
                sidx[pl.ds(j, _LANES)] = src[pl.ds(j, _LANES)] + offs

            pltpu.sync_copy(table_hbm.at[sidx], o_vmem)

        pltpu.emit_pipeline(
            body,
            grid=(N // _WINDOW,),
            in_specs=[pl.BlockSpec((1, _WINDOW), index_map=lambda i: (0, i))],
            out_specs=[pl.BlockSpec((_WINDOW, D), index_map=lambda i: (i, 0))],
            core_axis_name=("core", "subcore"),
            dimension_semantics=(pltpu.PARALLEL,),
        )(ids_hbm, out_hbm)

    out = gather_kernel(ids_flat, ttable)
    return out.reshape(B, S, H, D)


# speedup vs baseline: 1.2704x; 1.2704x over previous
import functools

import jax
import jax.numpy as jnp
from jax import lax
from jax.experimental import pallas as pl
from jax.experimental.pallas import tpu as pltpu
from jax.experimental.pallas import tpu_sc as plsc

_LANES = 16
_WINDOW = 512


def _transpose_table(tbl4, rows, d):
    tj_total = tbl4.shape[1]
    mesh = plsc.VectorSubcoreMesh(
        core_axis_name="core", subcore_axis_name="subcore"
    )
    n_workers = 32
    full_rounds = tj_total // n_workers
    tail = tj_total - full_rounds * n_workers

    @functools.partial(
        pl.kernel,
        out_type=jax.ShapeDtypeStruct((rows * d,), jnp.float32),
        mesh=mesh,
        scratch_types=[
            pltpu.VMEM((2, 8, 8, 128), jnp.float32),
            pltpu.VMEM((2, 128 * 64), jnp.float32),
            pltpu.SemaphoreType.DMA,
            pltpu.SemaphoreType.DMA,
            pltpu.SemaphoreType.DMA,
            pltpu.SemaphoreType.DMA,
        ],
        compiler_params=pltpu.CompilerParams(
            use_tc_tiling_on_sc=False, needs_layout_passes=False
        ),
    )
    def transpose_kernel(tbl4_hbm, out_hbm, vin, vout,
                         sem_i0, sem_i1, sem_o0, sem_o1):
        w = lax.axis_index("subcore") * 2 + lax.axis_index("core")
        rows16 = [k * _LANES + lax.iota(jnp.int32, _LANES) for k in range(8)]

        n_steps = full_rounds + (1 if tail else 0)
        assert n_steps % 2 == 0
        sem_i = [sem_i0, sem_i1]
        sem_o = [sem_o0, sem_o1]

        def tj_of(t):
            return t * n_workers + w

        def active(t):
            return (t < full_rounds) | (w < tail)

        def start_in(t, slot):
            pltpu.async_copy(tbl4_hbm.at[:, tj_of(t)], vin.at[slot],
                             sem_i[slot])

        def wait_in(t, slot):
            pltpu.make_async_copy(
                tbl4_hbm.at[:, tj_of(t)], vin.at[slot], sem_i[slot]
            ).wait()

        def start_out(t, slot):
            pltpu.async_copy(
                vout.at[slot],
                out_hbm.at[pl.ds(tj_of(t) * (128 * 64), 128 * 64)],
                sem_o[slot],
            )

        def wait_out(t, slot):
            pltpu.make_async_copy(
                vout.at[slot],
                out_hbm.at[pl.ds(tj_of(t) * (128 * 64), 128 * 64)],
                sem_o[slot],
            ).wait()

        rowbase = [r * 64 for r in rows16]

        def compute(slot):
            for k in range(8):
                base = rowbase[k]
                for dd in range(64):
                    vals = vin[slot, dd // 8, dd % 8, pl.ds(k * _LANES, _LANES)]
                    plsc.store_scatter(vout.at[slot], [base + dd], vals)

        start_in(0, 0)

        @pl.loop(0, n_steps // 2)
        def _(u):
            t0 = 2 * u
            t1 = 2 * u + 1

            wait_in(t0, 0)

            @pl.when(active(t1))
            def _():
                start_in(t1, 1)

            @pl.when(u >= 1)
            def _():
                wait_out(t0 - 2, 0)

            compute(0)
            start_out(t0, 0)

            @pl.when(active(t1))
            def _():
                wait_in(t1, 1)

                @pl.when((t1 + 1 < n_steps) & active(t1 + 1))
                def _():
                    start_in(t1 + 1, 0)

                @pl.when(u >= 1)
                def _():
                    wait_out(t1 - 2, 1)

                compute(1)
                start_out(t1, 1)

        wait_out(n_steps - 2, 0)

        @pl.when(active(n_steps - 1))
        def _():
            wait_out(n_steps - 1, 1)

    return transpose_kernel(tbl4)


def kernel(input_ids, table):
    B, S, H = input_ids.shape
    D = table.shape[-1]
    n_per_head = table.shape[0] // H
    N = B * S * H
    ids_flat = input_ids.reshape(1, N)

    tbl4 = jnp.transpose(
        table.reshape(table.shape[0] // 128, 128, 8, 8), (2, 0, 3, 1)
    )
    ttable = _transpose_table(tbl4, table.shape[0], D).reshape(
        table.shape[0], D
    )

    mesh = plsc.VectorSubcoreMesh(
        core_axis_name="core", subcore_axis_name="subcore"
    )

    @functools.partial(
        pl.kernel,
        out_type=jax.ShapeDtypeStruct((N, D), jnp.float32),
        mesh=mesh,
        scratch_types=[pltpu.VMEM((_WINDOW,), jnp.int32)],
        compiler_params=pltpu.CompilerParams(use_tc_tiling_on_sc=False),
    )
    def gather_kernel(ids_hbm, table_hbm, out_hbm, sidx):
        def body(i_vmem, o_vmem):
            offs = (
                lax.rem(
                    lax.iota(jnp.int32, _LANES),
                    jnp.full((_LANES,), H, jnp.int32),
                )
                * n_per_head
            )
            src = i_vmem.at[0]

            @pl.loop(0, _WINDOW, step=_LANES)
            def _(j):
                sidx[pl.ds(j, _LANES)] = src[pl.ds(j, _LANES)] + offs

            pltpu.sync_copy(table_hbm.at[sidx], o_vmem)

        pltpu.emit_pipeline(
            body,
            grid=(N // _WINDOW,),
            in_specs=[pl.BlockSpec((1, _WINDOW), index_map=lambda i: (0, i))],
            out_specs=[pl.BlockSpec((_WINDOW, D), index_map=lambda i: (i, 0))],
            core_axis_name=("core", "subcore"),
            dimension_semantics=(pltpu.PARALLEL,),
        )(ids_hbm, out_hbm)

    out = gather_kernel(ids_flat, ttable)
    return out.reshape(B, S, H, D)


# speedup vs baseline: 1.5890x; 1.2508x over previous
import functools

import jax
import jax.numpy as jnp
from jax import lax
from jax.experimental import pallas as pl
from jax.experimental.pallas import tpu as pltpu
from jax.experimental.pallas import tpu_sc as plsc

_LANES = 16
_WINDOW = 512


def _transpose_table(tbl4, rows, d):
    tj_total = tbl4.shape[1]
    mesh = plsc.VectorSubcoreMesh(
        core_axis_name="core", subcore_axis_name="subcore"
    )
    n_workers = 32
    full_rounds = tj_total // n_workers
    tail = tj_total - full_rounds * n_workers

    @functools.partial(
        pl.kernel,
        out_type=jax.ShapeDtypeStruct((rows * d,), jnp.float32),
        mesh=mesh,
        scratch_types=[
            pltpu.VMEM((2, 8, 8, 128), jnp.float32),
            pltpu.VMEM((2, 128 * 64), jnp.float32),
            pltpu.SemaphoreType.DMA,
            pltpu.SemaphoreType.DMA,
            pltpu.SemaphoreType.DMA,
            pltpu.SemaphoreType.DMA,
        ],
        compiler_params=pltpu.CompilerParams(
            use_tc_tiling_on_sc=False, needs_layout_passes=False
        ),
    )
    def transpose_kernel(tbl4_hbm, out_hbm, vin, vout,
                         sem_i0, sem_i1, sem_o0, sem_o1):
        w = lax.axis_index("subcore") * 2 + lax.axis_index("core")
        rows16 = [k * _LANES + lax.iota(jnp.int32, _LANES) for k in range(8)]

        n_steps = full_rounds + (1 if tail else 0)
        assert n_steps % 2 == 0
        sem_i = [sem_i0, sem_i1]
        sem_o = [sem_o0, sem_o1]

        def tj_of(t):
            return t * n_workers + w

        def active(t):
            return (t < full_rounds) | (w < tail)

        def start_in(t, slot):
            pltpu.async_copy(tbl4_hbm.at[:, tj_of(t)], vin.at[slot],
                             sem_i[slot])

        def wait_in(t, slot):
            pltpu.make_async_copy(
                tbl4_hbm.at[:, tj_of(t)], vin.at[slot], sem_i[slot]
            ).wait()

        def start_out(t, slot):
            pltpu.async_copy(
                vout.at[slot],
                out_hbm.at[pl.ds(tj_of(t) * (128 * 64), 128 * 64)],
                sem_o[slot],
            )

        def wait_out(t, slot):
            pltpu.make_async_copy(
                vout.at[slot],
                out_hbm.at[pl.ds(tj_of(t) * (128 * 64), 128 * 64)],
                sem_o[slot],
            ).wait()

        rowbase = [r * 64 for r in rows16]

        def compute(slot):
            for k in range(8):
                base = rowbase[k]
                for d0 in range(0, 64, 8):
                    vals = [
                        vin[slot, (d0 + j) // 8, (d0 + j) % 8,
                            pl.ds(k * _LANES, _LANES)]
                        for j in range(8)
                    ]
                    for j in range(8):
                        plsc.store_scatter(
                            vout.at[slot], [base + d0 + j], vals[j]
                        )

        start_in(0, 0)

        @pl.loop(0, n_steps // 2)
        def _(u):
            t0 = 2 * u
            t1 = 2 * u + 1

            wait_in(t0, 0)

            @pl.when(active(t1))
            def _():
                start_in(t1, 1)

            @pl.when(u >= 1)
            def _():
                wait_out(t0 - 2, 0)

            compute(0)
            start_out(t0, 0)

            @pl.when(active(t1))
            def _():
                wait_in(t1, 1)

                @pl.when((t1 + 1 < n_steps) & active(t1 + 1))
                def _():
                    start_in(t1 + 1, 0)

                @pl.when(u >= 1)
                def _():
                    wait_out(t1 - 2, 1)

                compute(1)
                start_out(t1, 1)

        wait_out(n_steps - 2, 0)

        @pl.when(active(n_steps - 1))
        def _():
            wait_out(n_steps - 1, 1)

    return transpose_kernel(tbl4)


def kernel(input_ids, table):
    B, S, H = input_ids.shape
    D = table.shape[-1]
    n_per_head = table.shape[0] // H
    N = B * S * H
    ids_flat = input_ids.reshape(1, N)

    tbl4 = jnp.transpose(
        table.reshape(table.shape[0] // 128, 128, 8, 8), (2, 0, 3, 1)
    )
    ttable = _transpose_table(tbl4, table.shape[0], D).reshape(
        table.shape[0], D
    )

    mesh = plsc.VectorSubcoreMesh(
        core_axis_name="core", subcore_axis_name="subcore"
    )

    @functools.partial(
        pl.kernel,
        out_type=jax.ShapeDtypeStruct((N, D), jnp.float32),
        mesh=mesh,
        scratch_types=[pltpu.VMEM((_WINDOW,), jnp.int32)],
        compiler_params=pltpu.CompilerParams(use_tc_tiling_on_sc=False),
    )
    def gather_kernel(ids_hbm, table_hbm, out_hbm, sidx):
        def body(i_vmem, o_vmem):
            offs = (
                lax.rem(
                    lax.iota(jnp.int32, _LANES),
                    jnp.full((_LANES,), H, jnp.int32),
                )
                * n_per_head
            )
            src = i_vmem.at[0]

            @pl.loop(0, _WINDOW, step=_LANES)
            def _(j):
                sidx[pl.ds(j, _LANES)] = src[pl.ds(j, _LANES)] + offs

            pltpu.sync_copy(table_hbm.at[sidx], o_vmem)

        pltpu.emit_pipeline(
            body,
            grid=(N // _WINDOW,),
            in_specs=[pl.BlockSpec((1, _WINDOW), index_map=lambda i: (0, i))],
            out_specs=[pl.BlockSpec((_WINDOW, D), index_map=lambda i: (i, 0))],
            core_axis_name=("core", "subcore"),
            dimension_semantics=(pltpu.PARALLEL,),
        )(ids_hbm, out_hbm)

    out = gather_kernel(ids_flat, ttable)
    return out.reshape(B, S, H, D)


# speedup vs baseline: 1.6273x; 1.0241x over previous
import functools

import jax
import jax.numpy as jnp
from jax import lax
from jax.experimental import pallas as pl
from jax.experimental.pallas import tpu as pltpu
from jax.experimental.pallas import tpu_sc as plsc

_LANES = 16
_WINDOW = 512


_TJB = 2


def _transpose_table(tbl4, rows, d):
    tj_total = tbl4.shape[1]
    mesh = plsc.VectorSubcoreMesh(
        core_axis_name="core", subcore_axis_name="subcore"
    )
    n_workers = 32
    n_blocks = tj_total // _TJB
    assert n_blocks * _TJB == tj_total
    full_rounds = n_blocks // n_workers
    tail = n_blocks - full_rounds * n_workers
    blk_words = _TJB * 128 * 64

    @functools.partial(
        pl.kernel,
        out_type=jax.ShapeDtypeStruct((rows * d,), jnp.float32),
        mesh=mesh,
        scratch_types=[
            pltpu.VMEM((2, 8, _TJB, 8, 128), jnp.float32),
            pltpu.VMEM((2, blk_words), jnp.float32),
            pltpu.SemaphoreType.DMA,
            pltpu.SemaphoreType.DMA,
            pltpu.SemaphoreType.DMA,
            pltpu.SemaphoreType.DMA,
        ],
        compiler_params=pltpu.CompilerParams(
            use_tc_tiling_on_sc=False, needs_layout_passes=False
        ),
    )
    def transpose_kernel(tbl4_hbm, out_hbm, vin, vout,
                         sem_i0, sem_i1, sem_o0, sem_o1):
        w = lax.axis_index("subcore") * 2 + lax.axis_index("core")

        n_steps = full_rounds + (1 if tail else 0)
        assert n_steps % 2 == 0
        sem_i = [sem_i0, sem_i1]
        sem_o = [sem_o0, sem_o1]

        def blk_of(t):
            return t * n_workers + w

        def active(t):
            return (t < full_rounds) | (w < tail)

        def start_in(t, slot):
            pltpu.async_copy(
                tbl4_hbm.at[:, pl.ds(blk_of(t) * _TJB, _TJB)],
                vin.at[slot],
                sem_i[slot],
            )

        def wait_in(t, slot):
            pltpu.make_async_copy(
                tbl4_hbm.at[:, pl.ds(blk_of(t) * _TJB, _TJB)],
                vin.at[slot],
                sem_i[slot],
            ).wait()

        def start_out(t, slot):
            pltpu.async_copy(
                vout.at[slot],
                out_hbm.at[pl.ds(blk_of(t) * blk_words, blk_words)],
                sem_o[slot],
            )

        def wait_out(t, slot):
            pltpu.make_async_copy(
                vout.at[slot],
                out_hbm.at[pl.ds(blk_of(t) * blk_words, blk_words)],
                sem_o[slot],
            ).wait()

        def compute(slot):
            @pl.loop(0, 8)
            def _(k):
                rb = (k * _LANES + lax.iota(jnp.int32, _LANES)) * 64
                for cc in range(_TJB):
                    base = rb + cc * (128 * 64)
                    for d0 in range(0, 64, 8):
                        vals = [
                            vin[slot, (d0 + j) // 8, cc, (d0 + j) % 8,
                                pl.ds(k * _LANES, _LANES)]
                            for j in range(8)
                        ]
                        for j in range(8):
                            plsc.store_scatter(
                                vout.at[slot], [base + d0 + j], vals[j]
                            )

        start_in(0, 0)

        @pl.loop(0, n_steps // 2)
        def _(u):
            t0 = 2 * u
            t1 = 2 * u + 1

            wait_in(t0, 0)

            @pl.when(active(t1))
            def _():
                start_in(t1, 1)

            @pl.when(u >= 1)
            def _():
                wait_out(t0 - 2, 0)

            compute(0)
            start_out(t0, 0)

            @pl.when(active(t1))
            def _():
                wait_in(t1, 1)

                @pl.when((t1 + 1 < n_steps) & active(t1 + 1))
                def _():
                    start_in(t1 + 1, 0)

                @pl.when(u >= 1)
                def _():
                    wait_out(t1 - 2, 1)

                compute(1)
                start_out(t1, 1)

        wait_out(n_steps - 2, 0)

        @pl.when(active(n_steps - 1))
        def _():
            wait_out(n_steps - 1, 1)

    return transpose_kernel(tbl4)


def kernel(input_ids, table):
    B, S, H = input_ids.shape
    D = table.shape[-1]
    n_per_head = table.shape[0] // H
    N = B * S * H
    ids_flat = input_ids.reshape(1, N)

    tbl4 = jnp.transpose(
        table.reshape(table.shape[0] // 128, 128, 8, 8), (2, 0, 3, 1)
    )
    ttable = _transpose_table(tbl4, table.shape[0], D).reshape(
        table.shape[0], D
    )

    mesh = plsc.VectorSubcoreMesh(
        core_axis_name="core", subcore_axis_name="subcore"
    )

    @functools.partial(
        pl.kernel,
        out_type=jax.ShapeDtypeStruct((N, D), jnp.float32),
        mesh=mesh,
        scratch_types=[pltpu.VMEM((_WINDOW,), jnp.int32)],
        compiler_params=pltpu.CompilerParams(use_tc_tiling_on_sc=False),
    )
    def gather_kernel(ids_hbm, table_hbm, out_hbm, sidx):
        def body(i_vmem, o_vmem):
            offs = (
                lax.rem(
                    lax.iota(jnp.int32, _LANES),
                    jnp.full((_LANES,), H, jnp.int32),
                )
                * n_per_head
            )
            src = i_vmem.at[0]

            @pl.loop(0, _WINDOW, step=_LANES)
            def _(j):
                sidx[pl.ds(j, _LANES)] = src[pl.ds(j, _LANES)] + offs

            pltpu.sync_copy(table_hbm.at[sidx], o_vmem)

        pltpu.emit_pipeline(
            body,
            grid=(N // _WINDOW,),
            in_specs=[pl.BlockSpec((1, _WINDOW), index_map=lambda i: (0, i))],
            out_specs=[pl.BlockSpec((_WINDOW, D), index_map=lambda i: (i, 0))],
            core_axis_name=("core", "subcore"),
            dimension_semantics=(pltpu.PARALLEL,),
        )(ids_hbm, out_hbm)

    out = gather_kernel(ids_flat, ttable)
    return out.reshape(B, S, H, D)
